# baseline (speedup 1.0000x reference)
import functools

import jax
import jax.numpy as jnp
from jax import lax
from jax.experimental import pallas as pl
from jax.experimental.pallas import tpu as pltpu
from jax.experimental.pallas import tpu_sc as plsc

N = 10000
D_IN = 128
D_HID = 256
D_OUT = 128
K = 10
ALPHA = 0.1

NTILES = 32
NSUB = 16
NP = 10240
RPT = NP // NSUB
HRPT = RPT // 2
GROW = 10368

E = 320000
ET = E + N
CHUNK = 128
TCH = 168
IB = 24
NBLK = TCH // IB
EP = NSUB * TCH * CHUNK

NB = 400


WBC = 32


def _make_layer(W):
    def one_step(c, s, src_hbm, dst_hbm, s_in, c0_in, d2_hbm, s_out,
                 srcb, dstb, rows0, rows1, wb_v, d2_v, agg_sh,
                 gsem0, gsem1, ssem0, ssem1):
        pltpu.sync_copy(c0_in.at[pl.ds(c * NP + s * RPT, RPT)],
                        agg_sh.at[pl.ds(s * RPT, RPT)])
        plsc.subcore_barrier()

        def blk_body(blk, _):
            pltpu.sync_copy(src_hbm.at[c * NSUB + s, pl.ds(blk * IB, IB)],
                            srcb)
            pltpu.sync_copy(dst_hbm.at[s, pl.ds(blk * IB, IB)], dstb)
            pltpu.async_copy(s_in.at[srcb.at[0]], rows0, gsem0)

            def pair(p, _):
                j0 = 2 * p
                j1 = 2 * p + 1
                j2 = jnp.minimum(2 * p + 2, IB - 1)
                pltpu.make_async_copy(
                    s_in.at[srcb.at[j0]], rows0, gsem0).wait()
                pltpu.async_copy(s_in.at[srcb.at[j1]], rows1, gsem1)
                pltpu.async_copy(rows0, agg_sh.at[dstb.at[j0]], ssem0,
                                 add=True)
                pltpu.make_async_copy(
                    s_in.at[srcb.at[j1]], rows1, gsem1).wait()
                pltpu.make_async_copy(
                    rows0, agg_sh.at[dstb.at[j0]], ssem0).wait()
                pltpu.async_copy(s_in.at[srcb.at[j2]], rows0, gsem0)
                pltpu.async_copy(rows1, agg_sh.at[dstb.at[j1]], ssem1,
                                 add=True)
                pltpu.make_async_copy(
                    rows1, agg_sh.at[dstb.at[j1]], ssem1).wait()
                return 0

            lax.fori_loop(0, IB // 2, pair, 0)
            pltpu.make_async_copy(
                s_in.at[srcb.at[IB - 1]], rows0, gsem0).wait()
            return 0

        lax.fori_loop(0, NBLK, blk_body, 0)
        plsc.subcore_barrier()

        def body_wb(rep, _):
            base = s * RPT + rep * WBC
            pltpu.sync_copy(agg_sh.at[pl.ds(base, WBC)], wb_v)
            pltpu.sync_copy(d2_hbm.at[pl.ds(base, WBC)], d2_v)

            def body_r(r, _):
                dv = d2_v[r]
                for dc in range(W // 16):
                    sl = pl.ds(dc * 16, 16)
                    wb_v[r, sl] = wb_v[r, sl] * dv
                return 0

            lax.fori_loop(0, WBC, body_r, 0)
            pltpu.sync_copy(wb_v, s_out.at[pl.ds(c * NP + base, WBC)])
            return 0

        lax.fori_loop(0, RPT // WBC, body_wb, 0)

    def layer_body(src_hbm, dst_hbm, s_in, c0_in, d2_hbm, bufs,
                   srcb, dstb, rows0, rows1, wb_v, d2_v, agg_sh,
                   gsem0, gsem1, ssem0, ssem1):
        c = lax.axis_index("c")
        s = lax.axis_index("s")
        for k in range(K):
            sin = s_in if k == 0 else bufs.at[(k - 1) % 2]
            one_step(c, s, src_hbm, dst_hbm, sin, c0_in, d2_hbm,
                     bufs.at[k % 2], srcb, dstb, rows0, rows1, wb_v, d2_v,
                     agg_sh, gsem0, gsem1, ssem0, ssem1)

    return functools.partial(
        pl.kernel,
        mesh=plsc.VectorSubcoreMesh(core_axis_name="c", subcore_axis_name="s"),
        out_type=jax.ShapeDtypeStruct((2, 2 * NP, W), jnp.float32),
        scratch_types=[
            pltpu.VMEM((IB, CHUNK), jnp.int32),
            pltpu.VMEM((IB, CHUNK), jnp.int32),
            pltpu.VMEM((CHUNK, W), jnp.float32),
            pltpu.VMEM((CHUNK, W), jnp.float32),
            pltpu.VMEM((WBC, W), jnp.float32),
            pltpu.VMEM((WBC, 16), jnp.float32),
            pltpu.VMEM_SHARED((GROW, W), jnp.float32),
            pltpu.SemaphoreType.DMA,
            pltpu.SemaphoreType.DMA,
            pltpu.SemaphoreType.DMA,
            pltpu.SemaphoreType.DMA,
        ],
    )(layer_body)


_layer128 = _make_layer(128)

TCHH = 88
IB2 = 8
NBLK2 = TCHH // IB2
EP2 = 2 * NSUB * TCHH * CHUNK


def _l2_body(src_hbm, dst_hbm, pbuf_in, tele, d2_hbm, pbuf_out, scop,
             srcb, dstb, rows0, rows1, wb_v, p1_v, d2_v, agg_sh,
             gsem0, gsem1, ssem0, ssem1):
    c = lax.axis_index("c")
    s = lax.axis_index("s")

    def body_cmb(rep, _):
        base = s * RPT + rep * WBC
        pltpu.sync_copy(pbuf_in.at[pl.ds(base, WBC)], wb_v)
        pltpu.sync_copy(pbuf_in.at[pl.ds(NP + base, WBC)], p1_v)
        pltpu.sync_copy(d2_hbm.at[pl.ds(base, WBC)], d2_v)

        def body_r(r, _):
            dv = d2_v[r]
            for dc in range(128 // 16):
                sl = pl.ds(dc * 16, 16)
                wb_v[r, sl] = (wb_v[r, sl] + p1_v[r, sl]) * dv
            return 0

        lax.fori_loop(0, WBC, body_r, 0)
        pltpu.sync_copy(wb_v, scop.at[pl.ds(c * NP + base, WBC)])
        return 0

    lax.fori_loop(0, RPT // WBC, body_cmb, 0)

    pltpu.sync_copy(tele.at[pl.ds(c * NP + s * RPT, RPT)],
                    agg_sh.at[pl.ds(s * RPT, RPT)])
    plsc.subcore_barrier()

    def blk_body(blk, _):
        pltpu.sync_copy(src_hbm.at[c * NSUB + s, pl.ds(blk * IB2, IB2)],
                        srcb)
        pltpu.sync_copy(dst_hbm.at[c * NSUB + s, pl.ds(blk * IB2, IB2)],
                        dstb)
        pltpu.async_copy(scop.at[srcb.at[0]], rows0, gsem0)

        def pair(p, _):
            j0 = 2 * p
            j1 = 2 * p + 1
            j2 = jnp.minimum(2 * p + 2, IB2 - 1)
            pltpu.make_async_copy(scop.at[srcb.at[j0]], rows0, gsem0).wait()
            pltpu.async_copy(scop.at[srcb.at[j1]], rows1, gsem1)
            pltpu.async_copy(rows0, agg_sh.at[dstb.at[j0]], ssem0, add=True)
            pltpu.make_async_copy(scop.at[srcb.at[j1]], rows1, gsem1).wait()
            pltpu.make_async_copy(rows0, agg_sh.at[dstb.at[j0]], ssem0).wait()
            pltpu.async_copy(scop.at[srcb.at[j2]], rows0, gsem0)
            pltpu.async_copy(rows1, agg_sh.at[dstb.at[j1]], ssem1, add=True)
            pltpu.make_async_copy(rows1, agg_sh.at[dstb.at[j1]], ssem1).wait()
            return 0

        lax.fori_loop(0, IB2 // 2, pair, 0)
        pltpu.make_async_copy(scop.at[srcb.at[IB2 - 1]], rows0, gsem0).wait()
        return 0

    lax.fori_loop(0, NBLK2, blk_body, 0)
    plsc.subcore_barrier()

    pltpu.sync_copy(agg_sh.at[pl.ds(s * RPT, RPT)],
                    pbuf_out.at[pl.ds(c * NP + s * RPT, RPT)])


_l2step = functools.partial(
    pl.kernel,
    mesh=plsc.VectorSubcoreMesh(core_axis_name="c", subcore_axis_name="s"),
    out_type=[
        jax.ShapeDtypeStruct((2 * NP, 128), jnp.float32),
        jax.ShapeDtypeStruct((2 * NP, 128), jnp.float32),
    ],
    scratch_types=[
        pltpu.VMEM((IB2, CHUNK), jnp.int32),
        pltpu.VMEM((IB2, CHUNK), jnp.int32),
        pltpu.VMEM((CHUNK, 128), jnp.float32),
        pltpu.VMEM((CHUNK, 128), jnp.float32),
        pltpu.VMEM((WBC, 128), jnp.float32),
        pltpu.VMEM((WBC, 128), jnp.float32),
        pltpu.VMEM((WBC, 16), jnp.float32),
        pltpu.VMEM_SHARED((GROW, 128), jnp.float32),
        pltpu.SemaphoreType.DMA,
        pltpu.SemaphoreType.DMA,
        pltpu.SemaphoreType.DMA,
        pltpu.SemaphoreType.DMA,
    ],
)(_l2_body)


def _deg_body(dst_hbm, zeros_in, ones_in, bufs, dstb, rows0, agg_sh,
              ssem0, ssem1):
    c = lax.axis_index("c")
    s = lax.axis_index("s")
    pltpu.sync_copy(ones_in.at[pl.ds(0, CHUNK)], rows0)
    pltpu.sync_copy(zeros_in.at[pl.ds(s * RPT, RPT)],
                    agg_sh.at[pl.ds(s * RPT, RPT)])
    plsc.subcore_barrier()

    def blk_body(blk, _):
        pltpu.sync_copy(dst_hbm.at[s, pl.ds(blk * IB, IB)], dstb)

        def pair(p, _):
            j0 = 2 * p
            j1 = 2 * p + 1
            pltpu.async_copy(rows0, agg_sh.at[dstb.at[j0]], ssem0, add=True)
            pltpu.async_copy(rows0, agg_sh.at[dstb.at[j1]], ssem1, add=True)
            pltpu.make_async_copy(rows0, agg_sh.at[dstb.at[j0]], ssem0).wait()
            pltpu.make_async_copy(rows0, agg_sh.at[dstb.at[j1]], ssem1).wait()
            return 0

        lax.fori_loop(0, IB // 2, pair, 0)
        return 0

    lax.fori_loop(0, NBLK, blk_body, 0)
    plsc.subcore_barrier()

    def body_wb(rep, _):
        base = s * RPT + rep * WBC
        pltpu.sync_copy(agg_sh.at[pl.ds(base, WBC)],
                        bufs.at[pl.ds(c * NP + base, WBC)])
        return 0

    lax.fori_loop(0, RPT // WBC, body_wb, 0)


_deg = functools.partial(
    pl.kernel,
    mesh=plsc.VectorSubcoreMesh(core_axis_name="c", subcore_axis_name="s"),
    out_type=jax.ShapeDtypeStruct((2 * NP, 128), jnp.float32),
    scratch_types=[
        pltpu.VMEM((IB, CHUNK), jnp.int32),
        pltpu.VMEM((CHUNK, 128), jnp.float32),
        pltpu.VMEM_SHARED((GROW, 128), jnp.float32),
        pltpu.SemaphoreType.DMA,
        pltpu.SemaphoreType.DMA,
    ],
)(_deg_body)


def _tc1_body(x_ref, w_ref, b_ref, deg_ref, s0_ref, c0_ref, d2_ref):
    h = jnp.dot(x_ref[...], w_ref[...],
                preferred_element_type=jnp.float32) + b_ref[...]
    deg = deg_ref[...][:, 0:1]
    sq = jnp.sqrt(deg)
    s0 = h / sq
    c0 = (ALPHA / (1.0 - ALPHA)) * h * sq
    s0_ref[0] = s0[:, :128]
    s0_ref[1] = s0[:, 128:]
    c0_ref[0] = c0[:, :128]
    c0_ref[1] = c0[:, 128:]
    d2_ref[...] = (1.0 - ALPHA) / jnp.maximum(deg_ref[...], 1.0)


def _tc1(x, W1, b1, deg):
    return pl.pallas_call(
        _tc1_body,
        grid=(N // NB,),
        in_specs=[
            pl.BlockSpec((NB, D_IN), lambda i: (i, 0)),
            pl.BlockSpec((D_IN, D_HID), lambda i: (0, 0)),
            pl.BlockSpec((1, D_HID), lambda i: (0, 0)),
            pl.BlockSpec((NB, 16), lambda i: (i, 0)),
        ],
        out_specs=[
            pl.BlockSpec((2, NB, 128), lambda i: (0, i, 0)),
            pl.BlockSpec((2, NB, 128), lambda i: (0, i, 0)),
            pl.BlockSpec((NB, 16), lambda i: (i, 0)),
        ],
        out_shape=[
            jax.ShapeDtypeStruct((2, N, 128), jnp.float32),
            jax.ShapeDtypeStruct((2, N, 128), jnp.float32),
            jax.ShapeDtypeStruct((N, 16), jnp.float32),
        ],
    )(x, W1, b1, deg)


def _tc2_body(s_ref, w_ref, b_ref, deg_ref, s0_ref, c0_ref):
    deg = deg_ref[...][:, 0:1]
    sq = jnp.sqrt(deg)
    h = jnp.concatenate([s_ref[0], s_ref[1]], axis=1) * sq
    x2 = jnp.maximum(h, 0.0)
    h0 = jnp.dot(x2, w_ref[...], preferred_element_type=jnp.float32) + b_ref[...]
    s0 = h0 / sq
    c0 = (ALPHA / (1.0 - ALPHA)) * h0 * sq
    s0_ref[...] = s0 * jnp.maximum(deg, 1.0) / (1.0 - ALPHA)
    c0_ref[...] = c0


def _tc2(sK, W2, b2, deg):
    return pl.pallas_call(
        _tc2_body,
        grid=(N // NB,),
        in_specs=[
            pl.BlockSpec((2, NB, 128), lambda i: (0, i, 0)),
            pl.BlockSpec((D_HID, D_OUT), lambda i: (0, 0)),
            pl.BlockSpec((1, D_OUT), lambda i: (0, 0)),
            pl.BlockSpec((NB, 16), lambda i: (i, 0)),
        ],
        out_specs=[
            pl.BlockSpec((NB, D_OUT), lambda i: (i, 0)),
            pl.BlockSpec((NB, D_OUT), lambda i: (i, 0)),
        ],
        out_shape=[
            jax.ShapeDtypeStruct((N, D_OUT), jnp.float32),
            jax.ShapeDtypeStruct((N, D_OUT), jnp.float32),
        ],
    )(sK, W2, b2, deg)


def _tc3_body(p_ref, deg_ref, out_ref):
    deg = deg_ref[...][:, 0:1]
    d2 = (1.0 - ALPHA) / jnp.maximum(deg, 1.0)
    out_ref[...] = (p_ref[0] + p_ref[1]) * d2 * jnp.sqrt(deg)


def _tc3(part, deg):
    return pl.pallas_call(
        _tc3_body,
        grid=(N // NB,),
        in_specs=[
            pl.BlockSpec((2, NB, 128), lambda i: (0, i, 0)),
            pl.BlockSpec((NB, 16), lambda i: (i, 0)),
        ],
        out_specs=pl.BlockSpec((NB, D_OUT), lambda i: (i, 0)),
        out_shape=jax.ShapeDtypeStruct((N, D_OUT), jnp.float32),
    )(part, deg)


def _pad_state(a):
    w = a.shape[-1]
    a = jnp.pad(a, ((0, 0), (0, NP - N), (0, 0)))
    return a.reshape(2 * NP, w)


def kernel(x, edge_index, W1, b1, W2, b2):
    loop = jnp.arange(N, dtype=jnp.int32)
    src = jnp.concatenate([edge_index[0], loop,
                           jnp.zeros((EP - ET,), jnp.int32)])
    dst = jnp.concatenate([edge_index[1], loop,
                           jnp.full((EP - ET,), NP, jnp.int32)])
    src2 = jnp.stack([src, src + NP]).reshape(2 * NSUB, TCH, CHUNK)
    dst_t = dst.reshape(NSUB, TCH, CHUNK)

    ones_r = jnp.ones((CHUNK, 128), jnp.float32)
    zeros_np = jnp.zeros((NP, 128), jnp.float32)
    deg_p = _deg(dst_t, zeros_np, ones_r)
    deg = deg_p[:N, :16]

    s0, c0, d2 = _tc1(x, W1, b1.reshape(1, D_HID), deg)
    d2_p = jnp.concatenate([d2, jnp.ones((NP - N, 16), jnp.float32)])
    s = _layer128(src2, dst_t, _pad_state(s0), _pad_state(c0),
                  d2_p)[(K - 1) % 2]

    sK = s.reshape(2, NP, 128)[:, :N]
    p_init, c0b = _tc2(sK, W2, b2.reshape(1, D_OUT), deg)

    src_h = jnp.concatenate([edge_index[0], loop,
                             jnp.zeros((EP2 - ET,), jnp.int32)])
    dst_h = jnp.concatenate([edge_index[1], loop,
                             jnp.full((EP2 - ET,), NP, jnp.int32)])
    src3 = (src_h.reshape(2, EP2 // 2)
            + jnp.array([[0], [NP]], jnp.int32)).reshape(
                2 * NSUB, TCHH, CHUNK)
    dst3 = dst_h.reshape(2 * NSUB, TCHH, CHUNK)

    zpad = jnp.zeros((NP, 128), jnp.float32)
    pbuf = jnp.concatenate([jnp.pad(p_init, ((0, NP - N), (0, 0))), zpad])
    tele = jnp.concatenate([jnp.pad(c0b, ((0, NP - N), (0, 0))), zpad])
    for _ in range(K):
        pbuf, _scop = _l2step(src3, dst3, pbuf, tele, d2_p)

    return _tc3(pbuf.reshape(2, NP, 128)[:, :N], deg)


# speedup vs baseline: 1.3573x; 1.3573x over previous
import functools

import jax
import jax.numpy as jnp
from jax import lax
from jax.experimental import pallas as pl
from jax.experimental.pallas import tpu as pltpu
from jax.experimental.pallas import tpu_sc as plsc

N = 10000
D_IN = 128
D_HID = 256
D_OUT = 128
K = 10
ALPHA = 0.1

NTILES = 32
NSUB = 16
NP = 10240
RPT = NP // NSUB
GROW = 10368

E = 320000
ET = E + N
CHUNK = 128
TCH = 168
IB = 24
NBLK = TCH // IB
EP = NSUB * TCH * CHUNK

NB = 400
WBC = 32


def _one_step(c, s, src_hbm, dst_hbm, s_in, c0_in, d2_hbm, s_out,
              srcb, dstb, rows0, rows1, agg_sh, gsem0, gsem1, ssem0, ssem1):
    pltpu.sync_copy(c0_in.at[pl.ds(c * NP + s * RPT, RPT)],
                    agg_sh.at[pl.ds(s * RPT, RPT)])
    plsc.subcore_barrier()

    def blk_body(blk, _):
        pltpu.sync_copy(src_hbm.at[c * NSUB + s, pl.ds(blk * IB, IB)], srcb)
        pltpu.sync_copy(dst_hbm.at[s, pl.ds(blk * IB, IB)], dstb)
        pltpu.async_copy(s_in.at[srcb.at[0]], rows0, gsem0)

        def pair(p, _):
            j0 = 2 * p
            j1 = 2 * p + 1
            j2 = jnp.minimum(2 * p + 2, IB - 1)
            pltpu.make_async_copy(s_in.at[srcb.at[j0]], rows0, gsem0).wait()
            pltpu.async_copy(s_in.at[srcb.at[j1]], rows1, gsem1)
            pltpu.async_copy(rows0, agg_sh.at[dstb.at[j0]], ssem0, add=True)
            pltpu.make_async_copy(s_in.at[srcb.at[j1]], rows1, gsem1).wait()
            pltpu.make_async_copy(rows0, agg_sh.at[dstb.at[j0]], ssem0).wait()
            pltpu.async_copy(s_in.at[srcb.at[j2]], rows0, gsem0)
            pltpu.async_copy(rows1, agg_sh.at[dstb.at[j1]], ssem1, add=True)
            pltpu.make_async_copy(rows1, agg_sh.at[dstb.at[j1]], ssem1).wait()
            return 0

        lax.fori_loop(0, IB // 2, pair, 0)
        pltpu.make_async_copy(s_in.at[srcb.at[IB - 1]], rows0, gsem0).wait()
        return 0

    lax.fori_loop(0, NBLK, blk_body, 0)
    plsc.subcore_barrier()

    def body_wb(rep, _):
        base = s * RPT + rep * CHUNK
        pltpu.sync_copy(agg_sh.at[pl.ds(base, CHUNK)], rows0)
        pltpu.sync_copy(d2_hbm.at[pl.ds(base, CHUNK)], rows1)

        def body_r(r, _):
            for dc in range(128 // 16):
                sl = pl.ds(dc * 16, 16)
                rows0[r, sl] = rows0[r, sl] * rows1[r, sl]
            return 0

        lax.fori_loop(0, CHUNK, body_r, 0)
        pltpu.sync_copy(rows0, s_out.at[pl.ds(c * NP + base, CHUNK)])
        return 0

    lax.fori_loop(0, RPT // CHUNK, body_wb, 0)


def _layer_body(src_hbm, dst_hbm, s_in, c0_in, d2_hbm, bufs,
                srcb, dstb, rows0, rows1, agg_sh,
                gsem0, gsem1, ssem0, ssem1):
    c = lax.axis_index("c")
    s = lax.axis_index("s")
    for k in range(K):
        sin = s_in if k == 0 else bufs.at[(k - 1) % 2]
        _one_step(c, s, src_hbm, dst_hbm, sin, c0_in, d2_hbm,
                  bufs.at[k % 2], srcb, dstb, rows0, rows1,
                  agg_sh, gsem0, gsem1, ssem0, ssem1)


_layer = functools.partial(
    pl.kernel,
    mesh=plsc.VectorSubcoreMesh(core_axis_name="c", subcore_axis_name="s"),
    out_type=jax.ShapeDtypeStruct((2, 2 * NP, 128), jnp.float32),
    scratch_types=[
        pltpu.VMEM((IB, CHUNK), jnp.int32),
        pltpu.VMEM((IB, CHUNK), jnp.int32),
        pltpu.VMEM((CHUNK, 128), jnp.float32),
        pltpu.VMEM((CHUNK, 128), jnp.float32),
        pltpu.VMEM_SHARED((GROW, 128), jnp.float32),
        pltpu.SemaphoreType.DMA,
        pltpu.SemaphoreType.DMA,
        pltpu.SemaphoreType.DMA,
        pltpu.SemaphoreType.DMA,
    ],
)(_layer_body)


def _deg_body(dst_hbm, zeros_in, ones_in, bufs, dstb, rows0, agg_sh,
              ssem0, ssem1):
    c = lax.axis_index("c")
    s = lax.axis_index("s")
    pltpu.sync_copy(ones_in.at[pl.ds(0, CHUNK)], rows0)
    pltpu.sync_copy(zeros_in.at[pl.ds(s * RPT, RPT)],
                    agg_sh.at[pl.ds(s * RPT, RPT)])
    plsc.subcore_barrier()

    def blk_body(blk, _):
        pltpu.sync_copy(dst_hbm.at[s, pl.ds(blk * IB, IB)], dstb)

        def pair(p, _):
            j0 = 2 * p
            j1 = 2 * p + 1
            pltpu.async_copy(rows0, agg_sh.at[dstb.at[j0]], ssem0, add=True)
            pltpu.async_copy(rows0, agg_sh.at[dstb.at[j1]], ssem1, add=True)
            pltpu.make_async_copy(rows0, agg_sh.at[dstb.at[j0]], ssem0).wait()
            pltpu.make_async_copy(rows0, agg_sh.at[dstb.at[j1]], ssem1).wait()
            return 0

        lax.fori_loop(0, IB // 2, pair, 0)
        return 0

    lax.fori_loop(0, NBLK, blk_body, 0)
    plsc.subcore_barrier()

    def body_wb(rep, _):
        base = s * RPT + rep * WBC
        pltpu.sync_copy(agg_sh.at[pl.ds(base, WBC)],
                        bufs.at[pl.ds(c * NP + base, WBC)])
        return 0

    lax.fori_loop(0, RPT // WBC, body_wb, 0)


_deg = functools.partial(
    pl.kernel,
    mesh=plsc.VectorSubcoreMesh(core_axis_name="c", subcore_axis_name="s"),
    out_type=jax.ShapeDtypeStruct((2 * NP, 128), jnp.float32),
    scratch_types=[
        pltpu.VMEM((IB, CHUNK), jnp.int32),
        pltpu.VMEM((CHUNK, 128), jnp.float32),
        pltpu.VMEM_SHARED((GROW, 128), jnp.float32),
        pltpu.SemaphoreType.DMA,
        pltpu.SemaphoreType.DMA,
    ],
)(_deg_body)


def _tc1_body(x_ref, w_ref, b_ref, deg_ref, s0_ref, c0_ref, d2_ref):
    h = jnp.dot(x_ref[...], w_ref[...],
                preferred_element_type=jnp.float32) + b_ref[...]
    deg = deg_ref[...][:, 0:1]
    sq = jnp.sqrt(deg)
    s0 = h / sq
    c0 = (ALPHA / (1.0 - ALPHA)) * h * sq
    s0_ref[0] = s0[:, :128]
    s0_ref[1] = s0[:, 128:]
    c0_ref[0] = c0[:, :128]
    c0_ref[1] = c0[:, 128:]
    d2_ref[...] = (1.0 - ALPHA) / jnp.maximum(
        jnp.broadcast_to(deg, (NB, 128)), 1.0)


def _tc1(x, W1, b1, deg):
    return pl.pallas_call(
        _tc1_body,
        grid=(N // NB,),
        in_specs=[
            pl.BlockSpec((NB, D_IN), lambda i: (i, 0)),
            pl.BlockSpec((D_IN, D_HID), lambda i: (0, 0)),
            pl.BlockSpec((1, D_HID), lambda i: (0, 0)),
            pl.BlockSpec((NB, 16), lambda i: (i, 0)),
        ],
        out_specs=[
            pl.BlockSpec((2, NB, 128), lambda i: (0, i, 0)),
            pl.BlockSpec((2, NB, 128), lambda i: (0, i, 0)),
            pl.BlockSpec((NB, 128), lambda i: (i, 0)),
        ],
        out_shape=[
            jax.ShapeDtypeStruct((2, N, 128), jnp.float32),
            jax.ShapeDtypeStruct((2, N, 128), jnp.float32),
            jax.ShapeDtypeStruct((N, 128), jnp.float32),
        ],
    )(x, W1, b1, deg)


def _tc2_body(s_ref, w_ref, b_ref, deg_ref, s0_ref, c0_ref):
    deg = deg_ref[...][:, 0:1]
    sq = jnp.sqrt(deg)
    h = jnp.concatenate([s_ref[0], s_ref[1]], axis=1) * sq
    x2 = jnp.maximum(h, 0.0)
    h0 = jnp.dot(x2, w_ref[...], preferred_element_type=jnp.float32) + b_ref[...]
    s0 = h0 / sq
    c0 = (ALPHA / (1.0 - ALPHA)) * h0 * sq
    z = jnp.zeros((NB, 64), jnp.float32)
    s0_ref[0] = jnp.concatenate([s0[:, :64], z], axis=1)
    s0_ref[1] = jnp.concatenate([s0[:, 64:], z], axis=1)
    c0_ref[0] = jnp.concatenate([c0[:, :64], z], axis=1)
    c0_ref[1] = jnp.concatenate([c0[:, 64:], z], axis=1)


def _tc2(sK, W2, b2, deg):
    return pl.pallas_call(
        _tc2_body,
        grid=(N // NB,),
        in_specs=[
            pl.BlockSpec((2, NB, 128), lambda i: (0, i, 0)),
            pl.BlockSpec((D_HID, D_OUT), lambda i: (0, 0)),
            pl.BlockSpec((1, D_OUT), lambda i: (0, 0)),
            pl.BlockSpec((NB, 16), lambda i: (i, 0)),
        ],
        out_specs=[
            pl.BlockSpec((2, NB, 128), lambda i: (0, i, 0)),
            pl.BlockSpec((2, NB, 128), lambda i: (0, i, 0)),
        ],
        out_shape=[
            jax.ShapeDtypeStruct((2, N, 128), jnp.float32),
            jax.ShapeDtypeStruct((2, N, 128), jnp.float32),
        ],
    )(sK, W2, b2, deg)


def _tc3_body(s_ref, deg_ref, out_ref):
    sq = jnp.sqrt(deg_ref[...][:, 0:1])
    out_ref[...] = jnp.concatenate(
        [s_ref[0][:, :64], s_ref[1][:, :64]], axis=1) * sq


def _tc3(sK, deg):
    return pl.pallas_call(
        _tc3_body,
        grid=(N // NB,),
        in_specs=[
            pl.BlockSpec((2, NB, 128), lambda i: (0, i, 0)),
            pl.BlockSpec((NB, 16), lambda i: (i, 0)),
        ],
        out_specs=pl.BlockSpec((NB, D_OUT), lambda i: (i, 0)),
        out_shape=jax.ShapeDtypeStruct((N, D_OUT), jnp.float32),
    )(sK, deg)


def _pad_state(a):
    a = jnp.pad(a, ((0, 0), (0, NP - N), (0, 0)))
    return a.reshape(2 * NP, 128)


def kernel(x, edge_index, W1, b1, W2, b2):
    loop = jnp.arange(N, dtype=jnp.int32)
    src = jnp.concatenate([edge_index[0], loop,
                           jnp.zeros((EP - ET,), jnp.int32)])
    dst = jnp.concatenate([edge_index[1], loop,
                           jnp.full((EP - ET,), NP, jnp.int32)])
    src2 = jnp.stack([src, src + NP]).reshape(2 * NSUB, TCH, CHUNK)
    dst_t = dst.reshape(NSUB, TCH, CHUNK)

    ones_r = jnp.ones((CHUNK, 128), jnp.float32)
    zeros_np = jnp.zeros((NP, 128), jnp.float32)
    deg_p = _deg(dst_t, zeros_np, ones_r)
    deg = deg_p[:N, :16]

    s0, c0, d2 = _tc1(x, W1, b1.reshape(1, D_HID), deg)
    d2_p = jnp.concatenate([d2, jnp.ones((NP - N, 128), jnp.float32)])
    s = _layer(src2, dst_t, _pad_state(s0), _pad_state(c0),
               d2_p)[(K - 1) % 2]

    sK = s.reshape(2, NP, 128)[:, :N]
    s0b, c0b = _tc2(sK, W2, b2.reshape(1, D_OUT), deg)
    s = _layer(src2, dst_t, _pad_state(s0b), _pad_state(c0b),
               d2_p)[(K - 1) % 2]

    return _tc3(s.reshape(2, NP, 128)[:, :N], deg)
